# baseline (speedup 1.0000x reference)
import functools

import numpy as np
import jax
import jax.numpy as jnp
from jax import lax
from jax.experimental import pallas as pl
from jax.experimental.pallas import tpu as pltpu
from jax.experimental.pallas import tpu_sc as plsc

F32 = jnp.float32
DIM = 128
H = 8
DK = 16
NLAYERS = 2
NC = 2
NS = 16
NW = NC * NS
CB = 64

N_GENE = 10000
N_PATH_PAD = 512
N_VIRT_PAD = 16


def _proj_body(x_ref, w_ref, b_ref, *o_refs):
    x = x_ref[...]
    for p, o in enumerate(o_refs):
        o[...] = jnp.dot(x, w_ref[p], preferred_element_type=F32) + b_ref[p]


def _proj(x, ws, bs, rb):
    n = x.shape[0]
    P = ws.shape[0]
    grid = n // rb
    return pl.pallas_call(
        _proj_body,
        grid=(grid,),
        in_specs=[
            pl.BlockSpec((rb, DIM), lambda i: (i, 0)),
            pl.BlockSpec((P, DIM, DIM), lambda i: (0, 0, 0)),
            pl.BlockSpec((P, 1, DIM), lambda i: (0, 0, 0)),
        ],
        out_specs=[pl.BlockSpec((rb, DIM), lambda i: (i, 0))] * P,
        out_shape=[jax.ShapeDtypeStruct((n, DIM), F32)] * P,
    )(x, ws, bs.reshape(P, 1, DIM))


def _gene_init_body(expr_ref, embr_ref, w1_ref, b1_ref, w2_ref, b2_ref,
                    ga_ref, gb_ref, gbias_ref, o_ref):
    a = expr_ref[...] * w1_ref[...] + b1_ref[...]
    hx = 0.5 * a * (1.0 + lax.erf(a * np.float32(0.7071067811865476)))
    hx = jnp.dot(hx, w2_ref[...], preferred_element_type=F32) + b2_ref[...]
    o_ref[...] = (jnp.dot(hx, ga_ref[...], preferred_element_type=F32)
                  + jnp.dot(embr_ref[...], gb_ref[...], preferred_element_type=F32)
                  + gbias_ref[...])


def _gene_init(expr, emb_rows, w1, b1, w2, b2, gw, gb):
    n = expr.shape[0]
    rb = 1000
    grid = n // rb
    return pl.pallas_call(
        _gene_init_body,
        grid=(grid,),
        in_specs=[
            pl.BlockSpec((rb, 1), lambda i: (i, 0)),
            pl.BlockSpec((rb, 256), lambda i: (i, 0)),
            pl.BlockSpec((1, 256), lambda i: (0, 0)),
            pl.BlockSpec((1, 256), lambda i: (0, 0)),
            pl.BlockSpec((256, 256), lambda i: (0, 0)),
            pl.BlockSpec((1, 256), lambda i: (0, 0)),
            pl.BlockSpec((256, DIM), lambda i: (0, 0)),
            pl.BlockSpec((256, DIM), lambda i: (0, 0)),
            pl.BlockSpec((1, DIM), lambda i: (0, 0)),
        ],
        out_specs=pl.BlockSpec((rb, DIM), lambda i: (i, 0)),
        out_shape=jax.ShapeDtypeStruct((n, DIM), F32),
    )(expr, emb_rows, w1, b1.reshape(1, 256), w2, b2.reshape(1, 256),
      gw[:256], gw[256:], gb.reshape(1, DIM))


def _scalar_init_body(f_ref, w_ref, b_ref, o_ref):
    o_ref[...] = f_ref[...] * w_ref[...] + b_ref[...]


def _scalar_init(feat, w, b):
    n = feat.shape[0]
    return pl.pallas_call(
        _scalar_init_body,
        out_shape=jax.ShapeDtypeStruct((n, DIM), F32),
    )(feat.reshape(n, 1), w.reshape(1, DIM), b.reshape(1, DIM))


_R_HEAD = np.zeros((16, DIM), np.float32)
for _h in range(H):
    _R_HEAD[_h, _h * DK:(_h + 1) * DK] = 1.0


def _finish_body(h_ref, p0_ref, p1_ref, d0_ref, d1_ref, r_ref,
                 wo_ref, bo_ref, g_ref, b_ref, o_ref):
    den = jnp.dot(d0_ref[0] + d1_ref[0], r_ref[...],
                  preferred_element_type=F32)
    s = p0_ref[0] + p1_ref[0]
    attn = s / jnp.maximum(den, np.float32(1e-30))
    x = h_ref[...] + jnp.dot(attn, wo_ref[...], preferred_element_type=F32) + bo_ref[...]
    mu = jnp.mean(x, axis=-1, keepdims=True)
    xc = x - mu
    var = jnp.mean(xc * xc, axis=-1, keepdims=True)
    o_ref[...] = g_ref[...] * xc * lax.rsqrt(var + np.float32(1e-5)) + b_ref[...]


def _finish(h, p_out, p_den, wo, bo, g, b, rb):
    n = h.shape[0]
    grid = n // rb
    return pl.pallas_call(
        _finish_body,
        grid=(grid,),
        in_specs=[
            pl.BlockSpec((rb, DIM), lambda i: (i, 0)),
            pl.BlockSpec((1, rb, DIM), lambda i: (0, i, 0)),
            pl.BlockSpec((1, rb, DIM), lambda i: (1, i, 0)),
            pl.BlockSpec((1, rb, 16), lambda i: (0, i, 0)),
            pl.BlockSpec((1, rb, 16), lambda i: (1, i, 0)),
            pl.BlockSpec((16, DIM), lambda i: (0, 0)),
            pl.BlockSpec((DIM, DIM), lambda i: (0, 0)),
            pl.BlockSpec((1, DIM), lambda i: (0, 0)),
            pl.BlockSpec((1, DIM), lambda i: (0, 0)),
            pl.BlockSpec((1, DIM), lambda i: (0, 0)),
        ],
        out_specs=pl.BlockSpec((rb, DIM), lambda i: (i, 0)),
        out_shape=jax.ShapeDtypeStruct((n, DIM), F32),
    )(h, p_out, p_out, p_den, p_den, jnp.asarray(_R_HEAD),
      wo, bo.reshape(1, DIM), g.reshape(1, DIM), b.reshape(1, DIM))


def _shuffle16(x, p):
    dn = lax.GatherDimensionNumbers(offset_dims=(), collapsed_slice_dims=(0,),
                                    start_index_map=(0,))
    return lax.gather(x, p[:, None], dn, slice_sizes=(1,),
                      mode=lax.GatherScatterMode.PROMISE_IN_BOUNDS)


def _sc_mesh():
    return plsc.VectorSubcoreMesh(core_axis_name="c", subcore_axis_name="s",
                                  num_cores=NC, num_subcores=NS)


@functools.lru_cache(maxsize=None)
def _edge_pass_fn(epad, nd_sc, steps):
    rows_t = nd_sc // NS
    rows_td = rows_t // 8
    nsteps_zd = rows_t // CB

    @functools.partial(
        pl.kernel,
        out_type=[jax.ShapeDtypeStruct((NC, nd_sc, DIM), F32),
                  jax.ShapeDtypeStruct((NC, nd_sc, 16), F32)],
        mesh=_sc_mesh(),
        scratch_types=[
            pltpu.VMEM((CB,), jnp.int32),
            pltpu.VMEM((CB,), jnp.int32),
            pltpu.VMEM((CB,), jnp.int32),
            pltpu.VMEM((CB,), jnp.int32),
            pltpu.VMEM((CB, DIM), F32),
            pltpu.VMEM((CB, DIM), F32),
            pltpu.VMEM((CB, DIM), F32),
            pltpu.VMEM((CB, 16), F32),
            pltpu.VMEM_SHARED((nd_sc, DIM), F32),
            pltpu.VMEM_SHARED((nd_sc // 8, DIM), F32),
            pltpu.SemaphoreType.DMA,
        ],
    )
    def kern(q_hbm, k_hbm, v_hbm, src_hbm, dst_hbm, dsts_hbm,
             out_hbm, den_hbm,
             srcv, dstv, dstsv, d2v, kv, qv, vv, dvo, acc_o, acc_d, sem):
        cid = lax.axis_index("c")
        sid = lax.axis_index("s")
        wid = sid * NC + cid
        r0 = sid * rows_t
        r0d = sid * rows_td

        zvec = jnp.zeros((16,), F32)

        def zfill(j, c):
            kv[j >> 3, pl.ds((j & 7) * 16, 16)] = zvec
            return c

        lax.fori_loop(0, CB * 8, zfill, 0)

        def zstep(j, c):
            pltpu.sync_copy(kv, acc_o.at[pl.ds(r0 + j * CB, CB)])
            pltpu.sync_copy(kv.at[pl.ds(0, 8)], acc_d.at[pl.ds(r0d + j * 8, 8)])
            return c

        lax.fori_loop(0, nsteps_zd, zstep, 0)
        plsc.subcore_barrier()

        wbase = wid * (steps * CB)

        def step(t, carry):
            base = wbase + t * CB
            pltpu.sync_copy(src_hbm.at[pl.ds(base, CB)], srcv)
            pltpu.sync_copy(dst_hbm.at[pl.ds(base, CB)], dstv)
            pltpu.sync_copy(dsts_hbm.at[pl.ds(base, CB)], dstsv)
            for g in range(CB // 16):
                d2v[pl.ds(g * 16, 16)] = lax.shift_right_logical(
                    dstsv[pl.ds(g * 16, 16)], 3)
            pltpu.async_copy(k_hbm.at[srcv], kv, sem).wait()
            pltpu.async_copy(q_hbm.at[dstv], qv, sem).wait()
            pltpu.async_copy(v_hbm.at[srcv], vv, sem).wait()

            lane = lax.iota(jnp.int32, 16)
            perms = [jnp.bitwise_xor(lane, sh) for sh in (8, 4, 2, 1)]
            seven = jnp.full((16,), 7, jnp.int32)

            def edge(e, c2):
                den_row = jnp.zeros((16,), F32)
                for h in range(H):
                    kvec = kv[e, pl.ds(h * DK, DK)]
                    qvec = qv[e, pl.ds(h * DK, DK)]
                    x = kvec * qvec
                    for p in perms:
                        x = x + _shuffle16(x, p)
                    ev = jnp.exp(x * np.float32(0.25))
                    vv[e, pl.ds(h * DK, DK)] = vv[e, pl.ds(h * DK, DK)] * ev
                    den_row = den_row + jnp.where(lane == h, ev, np.float32(0.0))
                dgrp = dstsv[pl.ds((e >> 4) * 16, 16)]
                dlow = _shuffle16(dgrp, jnp.full((16,), e & 15, jnp.int32)) & seven
                df = dlow.astype(F32)
                for c in range(8):
                    w = jnp.maximum(np.float32(0.0),
                                    np.float32(1.0) - jnp.abs(df - np.float32(c)))
                    qv[e, pl.ds(c * 16, 16)] = den_row * w
                return c2

            lax.fori_loop(0, CB, edge, 0)
            pltpu.sync_copy(vv, acc_o.at[dstsv], add=True)
            pltpu.sync_copy(qv, acc_d.at[d2v], add=True)
            return carry

        lax.fori_loop(0, steps, step, 0)
        plsc.subcore_barrier()

        def dstep(j, c):
            pltpu.sync_copy(acc_o.at[pl.ds(r0 + j * CB, CB)], kv)
            pltpu.sync_copy(kv, out_hbm.at[cid, pl.ds(r0 + j * CB, CB)])
            pltpu.sync_copy(acc_d.at[pl.ds(r0d + j * 8, 8)], qv.at[pl.ds(0, 8)])
            for j8 in range(8):
                for j2 in range(8):
                    dvo[j8 * 8 + j2, pl.ds(0, 16)] = qv[j8, pl.ds(j2 * 16, 16)]
            pltpu.sync_copy(dvo, den_hbm.at[cid, pl.ds(r0 + j * CB, CB)])
            return c

        lax.fori_loop(0, nsteps_zd, dstep, 0)

    return kern


@functools.lru_cache(maxsize=None)
def _edge_msg_fn(epad, steps):
    @functools.partial(
        pl.kernel,
        out_type=[jax.ShapeDtypeStruct((epad, DIM), F32),
                  jax.ShapeDtypeStruct((epad, 16), F32)],
        mesh=_sc_mesh(),
        scratch_types=[
            pltpu.VMEM((CB,), jnp.int32),
            pltpu.VMEM((CB,), jnp.int32),
            pltpu.VMEM((CB, DIM), F32),
            pltpu.VMEM((CB, DIM), F32),
            pltpu.VMEM((CB, DIM), F32),
            pltpu.VMEM((CB, 16), F32),
            pltpu.SemaphoreType.DMA,
        ],
    )
    def kern(q_hbm, k_hbm, v_hbm, src_hbm, dst_hbm, msg_hbm, den_hbm,
             srcv, dstv, kv, qv, vv, dv, sem):
        cid = lax.axis_index("c")
        sid = lax.axis_index("s")
        wid = sid * NC + cid
        wbase = wid * (steps * CB)

        def step(t, carry):
            base = wbase + t * CB
            pltpu.sync_copy(src_hbm.at[pl.ds(base, CB)], srcv)
            pltpu.sync_copy(dst_hbm.at[pl.ds(base, CB)], dstv)
            pltpu.async_copy(k_hbm.at[srcv], kv, sem).wait()
            pltpu.async_copy(q_hbm.at[dstv], qv, sem).wait()
            pltpu.async_copy(v_hbm.at[srcv], vv, sem).wait()

            lane = lax.iota(jnp.int32, 16)
            perms = [jnp.bitwise_xor(lane, sh) for sh in (8, 4, 2, 1)]

            def edge(e, c2):
                den_row = jnp.zeros((16,), F32)
                for h in range(H):
                    kvec = kv[e, pl.ds(h * DK, DK)]
                    qvec = qv[e, pl.ds(h * DK, DK)]
                    x = kvec * qvec
                    for p in perms:
                        x = x + _shuffle16(x, p)
                    ev = jnp.exp(x * np.float32(0.25))
                    vv[e, pl.ds(h * DK, DK)] = vv[e, pl.ds(h * DK, DK)] * ev
                    den_row = den_row + jnp.where(lane == h, ev, np.float32(0.0))
                dv[e, pl.ds(0, 16)] = den_row
                return c2

            lax.fori_loop(0, CB, edge, 0)
            pltpu.sync_copy(vv, msg_hbm.at[pl.ds(base, CB)])
            pltpu.sync_copy(dv, den_hbm.at[pl.ds(base, CB)])
            return carry

        lax.fori_loop(0, steps, step, 0)

    return kern


def _edge_pass(q, k, v, src_pad, dstg_pad, dsts_pad, nd_sc, use_sc):
    if use_sc == 1:
        epad = src_pad.shape[0]
        steps = epad // (NW * CB)
        msg, den = _edge_msg_fn(epad, steps)(q, k, v, src_pad, dstg_pad)
        out = jax.ops.segment_sum(msg, dsts_pad, num_segments=nd_sc)
        dsum = jax.ops.segment_sum(den, dsts_pad, num_segments=nd_sc)
        return (jnp.stack([out, jnp.zeros_like(out)]),
                jnp.stack([dsum, jnp.zeros_like(dsum)]))
    if not use_sc:
        score = jnp.einsum("ehd,ehd->eh", k[src_pad].reshape(-1, H, DK),
                           q[dstg_pad].reshape(-1, H, DK)) * 0.25
        ev = jnp.exp(score)
        den = jax.ops.segment_sum(ev, dsts_pad, num_segments=nd_sc)
        msg = v[src_pad].reshape(-1, H, DK) * ev[:, :, None]
        out = jax.ops.segment_sum(msg, dsts_pad, num_segments=nd_sc).reshape(-1, DIM)
        zo = jnp.zeros_like(out)
        zd = jnp.zeros((nd_sc, 16), F32)
        return (jnp.stack([out, zo]),
                jnp.stack([jnp.concatenate([den, jnp.zeros((nd_sc, 8), F32)], 1), zd]))
    epad = src_pad.shape[0]
    steps = epad // (NW * CB)
    fn = _edge_pass_fn(epad, nd_sc, steps)
    return fn(q, k, v, src_pad, dstg_pad, dsts_pad)


@functools.lru_cache(maxsize=None)
def _emb_gather_fn(tpad, nvocab, dmodel):
    steps = tpad // (NW * CB)

    @functools.partial(
        pl.kernel,
        out_type=jax.ShapeDtypeStruct((tpad, dmodel), F32),
        mesh=_sc_mesh(),
        scratch_types=[
            pltpu.VMEM((CB,), jnp.int32),
            pltpu.VMEM((CB, dmodel), F32),
            pltpu.SemaphoreType.DMA,
        ],
    )
    def kern(emb_hbm, tok_hbm, out_hbm, idxv, rows, sem):
        cid = lax.axis_index("c")
        sid = lax.axis_index("s")
        wid = sid * NC + cid
        wbase = wid * (steps * CB)

        def step(t, carry):
            base = wbase + t * CB
            pltpu.sync_copy(tok_hbm.at[pl.ds(base, CB)], idxv)
            pltpu.async_copy(emb_hbm.at[idxv], rows, sem).wait()
            pltpu.sync_copy(rows, out_hbm.at[pl.ds(base, CB)])
            return carry

        lax.fori_loop(0, steps, step, 0)

    return kern


def _pad_edges(src, dst, trash_row):
    e_real = src.shape[0]
    chunk = NW * CB
    epad = ((e_real + chunk - 1) // chunk) * chunk
    src_p = jnp.zeros((epad,), jnp.int32).at[:e_real].set(src.astype(jnp.int32))
    dstg_p = jnp.zeros((epad,), jnp.int32).at[:e_real].set(dst.astype(jnp.int32))
    dsts_p = jnp.full((epad,), trash_row, jnp.int32).at[:e_real].set(dst.astype(jnp.int32))
    return src_p, dstg_p, dsts_p


def kernel(expr, token, path_feat, virt_feat, edge_g2g, src_g2p, dst_g2p,
           src_p2g, dst_p2g, edge_p2p, src_v2p, dst_v2p, src_p2v, dst_p2v,
           emb, mlp_W1, mlp_b1, mlp_W2, mlp_b2, gp_W, gp_b, pp_W, pp_b,
           vp_W, vp_b, Wq, bq, Wk, bk, Wv, bv, Wo, bo, ln_g, ln_b):
    n_gene = expr.shape[0]
    n_path = path_feat.shape[0]
    n_virt = virt_feat.shape[0]

    tpad = ((n_gene + NW * CB - 1) // (NW * CB)) * (NW * CB)
    tok_pad = jnp.zeros((tpad,), jnp.int32).at[:n_gene].set(token.astype(jnp.int32))
    emb_rows = _emb_gather_fn(tpad, emb.shape[0], emb.shape[1])(emb, tok_pad)[:n_gene]

    h = {
        "g": _gene_init(expr, emb_rows, mlp_W1, mlp_b1, mlp_W2, mlp_b2, gp_W, gp_b),
        "p": _scalar_init(jnp.zeros((N_PATH_PAD,), F32).at[:n_path].set(path_feat),
                          pp_W, pp_b),
        "v": _scalar_init(jnp.zeros((N_VIRT_PAD,), F32).at[:n_virt].set(virt_feat),
                          vp_W, vp_b),
    }
    nrows = {"g": n_gene, "p": N_PATH_PAD, "v": N_VIRT_PAD}
    rbs = {"g": 1000, "p": N_PATH_PAD, "v": N_VIRT_PAD}

    etypes = [
        ("g", "g", edge_g2g[0], edge_g2g[1]),
        ("g", "p", src_g2p, dst_g2p),
        ("p", "g", src_p2g, dst_p2g),
        ("p", "p", edge_p2p[0], edge_p2p[1]),
        ("v", "p", src_v2p, dst_v2p),
        ("p", "v", src_p2v, dst_p2v),
    ]
    padded = [_pad_edges(s, d, nrows[dk]) for (_, dk, s, d) in etypes]

    for l in range(NLAYERS):
        for t, (sk, dk, _, _) in enumerate(etypes):
            src_p, dstg_p, dsts_p = padded[t]
            if sk == dk:
                q, kk, vv = _proj(h[sk],
                                  jnp.stack([Wq[l, t], Wk[l, t], Wv[l, t]]),
                                  jnp.stack([bq[l, t], bk[l, t], bv[l, t]]),
                                  rbs[sk])
            else:
                (q,) = _proj(h[dk], Wq[l, t][None], bq[l, t][None], rbs[dk])
                kk, vv = _proj(h[sk],
                               jnp.stack([Wk[l, t], Wv[l, t]]),
                               jnp.stack([bk[l, t], bv[l, t]]),
                               rbs[sk])
            nd = nrows[dk]
            nd_sc = ((nd + 16 + NS * CB - 1) // (NS * CB)) * (NS * CB)
            p_out, p_den = _edge_pass(q, kk, vv, src_p, dstg_p, dsts_p, nd_sc,
                                      use_sc=2)
            h[dk] = _finish(h[dk], p_out, p_den, Wo[l, t],
                            bo[l, t], ln_g[l, t], ln_b[l, t], rbs[dk])

    return h["g"][:n_gene], h["p"][:n_path], h["v"][:n_virt]


# speedup vs baseline: 3.0146x; 3.0146x over previous
import functools

import numpy as np
import jax
import jax.numpy as jnp
from jax import lax
from jax.experimental import pallas as pl
from jax.experimental.pallas import tpu as pltpu
from jax.experimental.pallas import tpu_sc as plsc

F32 = jnp.float32
DIM = 128
H = 8
DK = 16
NLAYERS = 2
NC = 2
NS = 16
NW = NC * NS
CB = 64

N_GENE = 10000
N_PATH_PAD = 512
N_VIRT_PAD = 16


def _proj_body(x_ref, w_ref, b_ref, *o_refs):
    x = x_ref[...]
    for p, o in enumerate(o_refs):
        o[...] = jnp.dot(x, w_ref[p], preferred_element_type=F32) + b_ref[p]


def _proj(x, ws, bs, rb):
    n = x.shape[0]
    P = ws.shape[0]
    grid = n // rb
    return pl.pallas_call(
        _proj_body,
        grid=(grid,),
        in_specs=[
            pl.BlockSpec((rb, DIM), lambda i: (i, 0)),
            pl.BlockSpec((P, DIM, DIM), lambda i: (0, 0, 0)),
            pl.BlockSpec((P, 1, DIM), lambda i: (0, 0, 0)),
        ],
        out_specs=[pl.BlockSpec((rb, DIM), lambda i: (i, 0))] * P,
        out_shape=[jax.ShapeDtypeStruct((n, DIM), F32)] * P,
    )(x, ws, bs.reshape(P, 1, DIM))


def _gene_init_body(expr_ref, embr_ref, w1_ref, b1_ref, w2_ref, b2_ref,
                    ga_ref, gb_ref, gbias_ref, o_ref):
    a = expr_ref[...] * w1_ref[...] + b1_ref[...]
    hx = 0.5 * a * (1.0 + lax.erf(a * np.float32(0.7071067811865476)))
    hx = jnp.dot(hx, w2_ref[...], preferred_element_type=F32) + b2_ref[...]
    o_ref[...] = (jnp.dot(hx, ga_ref[...], preferred_element_type=F32)
                  + jnp.dot(embr_ref[...], gb_ref[...], preferred_element_type=F32)
                  + gbias_ref[...])


def _gene_init(expr, emb_rows, w1, b1, w2, b2, gw, gb):
    n = expr.shape[0]
    rb = 1000
    grid = n // rb
    return pl.pallas_call(
        _gene_init_body,
        grid=(grid,),
        in_specs=[
            pl.BlockSpec((rb, 1), lambda i: (i, 0)),
            pl.BlockSpec((rb, 256), lambda i: (i, 0)),
            pl.BlockSpec((1, 256), lambda i: (0, 0)),
            pl.BlockSpec((1, 256), lambda i: (0, 0)),
            pl.BlockSpec((256, 256), lambda i: (0, 0)),
            pl.BlockSpec((1, 256), lambda i: (0, 0)),
            pl.BlockSpec((256, DIM), lambda i: (0, 0)),
            pl.BlockSpec((256, DIM), lambda i: (0, 0)),
            pl.BlockSpec((1, DIM), lambda i: (0, 0)),
        ],
        out_specs=pl.BlockSpec((rb, DIM), lambda i: (i, 0)),
        out_shape=jax.ShapeDtypeStruct((n, DIM), F32),
    )(expr, emb_rows, w1, b1.reshape(1, 256), w2, b2.reshape(1, 256),
      gw[:256], gw[256:], gb.reshape(1, DIM))


def _scalar_init_body(f_ref, w_ref, b_ref, o_ref):
    o_ref[...] = f_ref[...] * w_ref[...] + b_ref[...]


def _scalar_init(feat, w, b):
    n = feat.shape[0]
    return pl.pallas_call(
        _scalar_init_body,
        out_shape=jax.ShapeDtypeStruct((n, DIM), F32),
    )(feat.reshape(n, 1), w.reshape(1, DIM), b.reshape(1, DIM))


_R_HEAD = np.zeros((16, DIM), np.float32)
for _h in range(H):
    _R_HEAD[_h, _h * DK:(_h + 1) * DK] = 1.0


def _finish_body(h_ref, p0_ref, p1_ref, d0_ref, d1_ref, r_ref,
                 wo_ref, bo_ref, g_ref, b_ref, o_ref):
    den = jnp.dot(d0_ref[0] + d1_ref[0], r_ref[...],
                  preferred_element_type=F32)
    s = p0_ref[0] + p1_ref[0]
    attn = s / jnp.maximum(den, np.float32(1e-30))
    x = h_ref[...] + jnp.dot(attn, wo_ref[...], preferred_element_type=F32) + bo_ref[...]
    mu = jnp.mean(x, axis=-1, keepdims=True)
    xc = x - mu
    var = jnp.mean(xc * xc, axis=-1, keepdims=True)
    o_ref[...] = g_ref[...] * xc * lax.rsqrt(var + np.float32(1e-5)) + b_ref[...]


def _finish(h, p_out, p_den, wo, bo, g, b, rb):
    n = h.shape[0]
    grid = n // rb
    return pl.pallas_call(
        _finish_body,
        grid=(grid,),
        in_specs=[
            pl.BlockSpec((rb, DIM), lambda i: (i, 0)),
            pl.BlockSpec((1, rb, DIM), lambda i: (0, i, 0)),
            pl.BlockSpec((1, rb, DIM), lambda i: (1, i, 0)),
            pl.BlockSpec((1, rb, 16), lambda i: (0, i, 0)),
            pl.BlockSpec((1, rb, 16), lambda i: (1, i, 0)),
            pl.BlockSpec((16, DIM), lambda i: (0, 0)),
            pl.BlockSpec((DIM, DIM), lambda i: (0, 0)),
            pl.BlockSpec((1, DIM), lambda i: (0, 0)),
            pl.BlockSpec((1, DIM), lambda i: (0, 0)),
            pl.BlockSpec((1, DIM), lambda i: (0, 0)),
        ],
        out_specs=pl.BlockSpec((rb, DIM), lambda i: (i, 0)),
        out_shape=jax.ShapeDtypeStruct((n, DIM), F32),
    )(h, p_out, p_out, p_den, p_den, jnp.asarray(_R_HEAD),
      wo, bo.reshape(1, DIM), g.reshape(1, DIM), b.reshape(1, DIM))


def _shuffle16(x, p):
    dn = lax.GatherDimensionNumbers(offset_dims=(), collapsed_slice_dims=(0,),
                                    start_index_map=(0,))
    return lax.gather(x, p[:, None], dn, slice_sizes=(1,),
                      mode=lax.GatherScatterMode.PROMISE_IN_BOUNDS)


def _sc_mesh():
    return plsc.VectorSubcoreMesh(core_axis_name="c", subcore_axis_name="s",
                                  num_cores=NC, num_subcores=NS)


@functools.lru_cache(maxsize=None)
def _edge_pass_fn(epad, nd_sc, steps):
    rows_t = nd_sc // NS
    rows_td = rows_t // 8
    nsteps_zd = rows_t // CB

    @functools.partial(
        pl.kernel,
        out_type=[jax.ShapeDtypeStruct((NC, nd_sc, DIM), F32),
                  jax.ShapeDtypeStruct((NC, nd_sc, 16), F32)],
        mesh=_sc_mesh(),
        scratch_types=[
            pltpu.VMEM((CB,), jnp.int32),
            pltpu.VMEM((CB,), jnp.int32),
            pltpu.VMEM((CB,), jnp.int32),
            pltpu.VMEM((CB,), jnp.int32),
            pltpu.VMEM((CB, DIM), F32),
            pltpu.VMEM((CB, DIM), F32),
            pltpu.VMEM((CB, DIM), F32),
            pltpu.VMEM((CB, 16), F32),
            pltpu.VMEM_SHARED((nd_sc, DIM), F32),
            pltpu.VMEM_SHARED((nd_sc // 8, DIM), F32),
            pltpu.SemaphoreType.DMA,
        ],
    )
    def kern(q_hbm, k_hbm, v_hbm, src_hbm, dst_hbm, dsts_hbm,
             out_hbm, den_hbm,
             srcv, dstv, dstsv, d2v, kv, qv, vv, dvo, acc_o, acc_d, sem):
        cid = lax.axis_index("c")
        sid = lax.axis_index("s")
        wid = sid * NC + cid
        r0 = sid * rows_t
        r0d = sid * rows_td

        zvec = jnp.zeros((16,), F32)

        def zfill(j, c):
            kv[j >> 3, pl.ds((j & 7) * 16, 16)] = zvec
            return c

        lax.fori_loop(0, CB * 8, zfill, 0)

        def zstep(j, c):
            pltpu.sync_copy(kv, acc_o.at[pl.ds(r0 + j * CB, CB)])
            pltpu.sync_copy(kv.at[pl.ds(0, 8)], acc_d.at[pl.ds(r0d + j * 8, 8)])
            return c

        lax.fori_loop(0, nsteps_zd, zstep, 0)
        plsc.subcore_barrier()

        wbase = wid * (steps * CB)

        def step(t, carry):
            base = wbase + t * CB
            c1 = pltpu.async_copy(src_hbm.at[pl.ds(base, CB)], srcv, sem)
            c2 = pltpu.async_copy(dst_hbm.at[pl.ds(base, CB)], dstv, sem)
            c3 = pltpu.async_copy(dsts_hbm.at[pl.ds(base, CB)], dstsv, sem)
            c1.wait(); c2.wait(); c3.wait()
            for g in range(CB // 16):
                d2v[pl.ds(g * 16, 16)] = lax.shift_right_logical(
                    dstsv[pl.ds(g * 16, 16)], 3)
            g1 = pltpu.async_copy(k_hbm.at[srcv], kv, sem)
            g2 = pltpu.async_copy(q_hbm.at[dstv], qv, sem)
            g3 = pltpu.async_copy(v_hbm.at[srcv], vv, sem)
            g1.wait(); g2.wait(); g3.wait()

            lane = lax.iota(jnp.int32, 16)
            perms = [jnp.bitwise_xor(lane, sh) for sh in (8, 4, 2, 1)]
            seven = jnp.full((16,), 7, jnp.int32)

            @plsc.parallel_loop(0, CB, 1)
            def edge(e):
                den_row = jnp.zeros((16,), F32)
                for h in range(H):
                    kvec = kv[e, pl.ds(h * DK, DK)]
                    qvec = qv[e, pl.ds(h * DK, DK)]
                    x = kvec * qvec
                    for p in perms:
                        x = x + _shuffle16(x, p)
                    ev = jnp.exp(x * np.float32(0.25))
                    vv[e, pl.ds(h * DK, DK)] = vv[e, pl.ds(h * DK, DK)] * ev
                    den_row = den_row + jnp.where(lane == h, ev, np.float32(0.0))
                dgrp = dstsv[pl.ds((e >> 4) * 16, 16)]
                dlow = _shuffle16(dgrp, jnp.full((16,), e & 15, jnp.int32)) & seven
                df = dlow.astype(F32)
                for c in range(8):
                    w = jnp.maximum(np.float32(0.0),
                                    np.float32(1.0) - jnp.abs(df - np.float32(c)))
                    qv[e, pl.ds(c * 16, 16)] = den_row * w

            pltpu.sync_copy(vv, acc_o.at[dstsv], add=True)
            pltpu.sync_copy(qv, acc_d.at[d2v], add=True)
            return carry

        lax.fori_loop(0, steps, step, 0)
        plsc.subcore_barrier()

        def dstep(j, c):
            pltpu.sync_copy(acc_o.at[pl.ds(r0 + j * CB, CB)], kv)
            pltpu.sync_copy(kv, out_hbm.at[cid, pl.ds(r0 + j * CB, CB)])
            pltpu.sync_copy(acc_d.at[pl.ds(r0d + j * 8, 8)], qv.at[pl.ds(0, 8)])
            for j8 in range(8):
                for j2 in range(8):
                    dvo[j8 * 8 + j2, pl.ds(0, 16)] = qv[j8, pl.ds(j2 * 16, 16)]
            pltpu.sync_copy(dvo, den_hbm.at[cid, pl.ds(r0 + j * CB, CB)])
            return c

        lax.fori_loop(0, nsteps_zd, dstep, 0)

    return kern


@functools.lru_cache(maxsize=None)
def _edge_msg_fn(epad, steps):
    @functools.partial(
        pl.kernel,
        out_type=[jax.ShapeDtypeStruct((epad, DIM), F32),
                  jax.ShapeDtypeStruct((epad, 16), F32)],
        mesh=_sc_mesh(),
        scratch_types=[
            pltpu.VMEM((CB,), jnp.int32),
            pltpu.VMEM((CB,), jnp.int32),
            pltpu.VMEM((CB, DIM), F32),
            pltpu.VMEM((CB, DIM), F32),
            pltpu.VMEM((CB, DIM), F32),
            pltpu.VMEM((CB, 16), F32),
            pltpu.SemaphoreType.DMA,
        ],
    )
    def kern(q_hbm, k_hbm, v_hbm, src_hbm, dst_hbm, msg_hbm, den_hbm,
             srcv, dstv, kv, qv, vv, dv, sem):
        cid = lax.axis_index("c")
        sid = lax.axis_index("s")
        wid = sid * NC + cid
        wbase = wid * (steps * CB)

        def step(t, carry):
            base = wbase + t * CB
            pltpu.sync_copy(src_hbm.at[pl.ds(base, CB)], srcv)
            pltpu.sync_copy(dst_hbm.at[pl.ds(base, CB)], dstv)
            pltpu.async_copy(k_hbm.at[srcv], kv, sem).wait()
            pltpu.async_copy(q_hbm.at[dstv], qv, sem).wait()
            pltpu.async_copy(v_hbm.at[srcv], vv, sem).wait()

            lane = lax.iota(jnp.int32, 16)
            perms = [jnp.bitwise_xor(lane, sh) for sh in (8, 4, 2, 1)]

            def edge(e, c2):
                den_row = jnp.zeros((16,), F32)
                for h in range(H):
                    kvec = kv[e, pl.ds(h * DK, DK)]
                    qvec = qv[e, pl.ds(h * DK, DK)]
                    x = kvec * qvec
                    for p in perms:
                        x = x + _shuffle16(x, p)
                    ev = jnp.exp(x * np.float32(0.25))
                    vv[e, pl.ds(h * DK, DK)] = vv[e, pl.ds(h * DK, DK)] * ev
                    den_row = den_row + jnp.where(lane == h, ev, np.float32(0.0))
                dv[e, pl.ds(0, 16)] = den_row
                return c2

            lax.fori_loop(0, CB, edge, 0)
            pltpu.sync_copy(vv, msg_hbm.at[pl.ds(base, CB)])
            pltpu.sync_copy(dv, den_hbm.at[pl.ds(base, CB)])
            return carry

        lax.fori_loop(0, steps, step, 0)

    return kern


def _edge_pass(q, k, v, src_pad, dstg_pad, dsts_pad, nd_sc, use_sc):
    if use_sc == 1:
        epad = src_pad.shape[0]
        steps = epad // (NW * CB)
        msg, den = _edge_msg_fn(epad, steps)(q, k, v, src_pad, dstg_pad)
        out = jax.ops.segment_sum(msg, dsts_pad, num_segments=nd_sc)
        dsum = jax.ops.segment_sum(den, dsts_pad, num_segments=nd_sc)
        return (jnp.stack([out, jnp.zeros_like(out)]),
                jnp.stack([dsum, jnp.zeros_like(dsum)]))
    if not use_sc:
        score = jnp.einsum("ehd,ehd->eh", k[src_pad].reshape(-1, H, DK),
                           q[dstg_pad].reshape(-1, H, DK)) * 0.25
        ev = jnp.exp(score)
        den = jax.ops.segment_sum(ev, dsts_pad, num_segments=nd_sc)
        msg = v[src_pad].reshape(-1, H, DK) * ev[:, :, None]
        out = jax.ops.segment_sum(msg, dsts_pad, num_segments=nd_sc).reshape(-1, DIM)
        zo = jnp.zeros_like(out)
        zd = jnp.zeros((nd_sc, 16), F32)
        return (jnp.stack([out, zo]),
                jnp.stack([jnp.concatenate([den, jnp.zeros((nd_sc, 8), F32)], 1), zd]))
    epad = src_pad.shape[0]
    steps = epad // (NW * CB)
    fn = _edge_pass_fn(epad, nd_sc, steps)
    return fn(q, k, v, src_pad, dstg_pad, dsts_pad)


@functools.lru_cache(maxsize=None)
def _emb_gather_fn(tpad, nvocab, dmodel):
    steps = tpad // (NW * CB)

    @functools.partial(
        pl.kernel,
        out_type=jax.ShapeDtypeStruct((tpad, dmodel), F32),
        mesh=_sc_mesh(),
        scratch_types=[
            pltpu.VMEM((CB,), jnp.int32),
            pltpu.VMEM((CB, dmodel), F32),
            pltpu.SemaphoreType.DMA,
        ],
    )
    def kern(emb_hbm, tok_hbm, out_hbm, idxv, rows, sem):
        cid = lax.axis_index("c")
        sid = lax.axis_index("s")
        wid = sid * NC + cid
        wbase = wid * (steps * CB)

        def step(t, carry):
            base = wbase + t * CB
            pltpu.sync_copy(tok_hbm.at[pl.ds(base, CB)], idxv)
            pltpu.async_copy(emb_hbm.at[idxv], rows, sem).wait()
            pltpu.sync_copy(rows, out_hbm.at[pl.ds(base, CB)])
            return carry

        lax.fori_loop(0, steps, step, 0)

    return kern


def _pad_edges(src, dst, trash_row):
    e_real = src.shape[0]
    chunk = NW * CB
    epad = ((e_real + chunk - 1) // chunk) * chunk
    src_p = jnp.zeros((epad,), jnp.int32).at[:e_real].set(src.astype(jnp.int32))
    dstg_p = jnp.zeros((epad,), jnp.int32).at[:e_real].set(dst.astype(jnp.int32))
    dsts_p = jnp.full((epad,), trash_row, jnp.int32).at[:e_real].set(dst.astype(jnp.int32))
    return src_p, dstg_p, dsts_p


def kernel(expr, token, path_feat, virt_feat, edge_g2g, src_g2p, dst_g2p,
           src_p2g, dst_p2g, edge_p2p, src_v2p, dst_v2p, src_p2v, dst_p2v,
           emb, mlp_W1, mlp_b1, mlp_W2, mlp_b2, gp_W, gp_b, pp_W, pp_b,
           vp_W, vp_b, Wq, bq, Wk, bk, Wv, bv, Wo, bo, ln_g, ln_b):
    n_gene = expr.shape[0]
    n_path = path_feat.shape[0]
    n_virt = virt_feat.shape[0]

    tpad = ((n_gene + NW * CB - 1) // (NW * CB)) * (NW * CB)
    tok_pad = jnp.zeros((tpad,), jnp.int32).at[:n_gene].set(token.astype(jnp.int32))
    emb_rows = _emb_gather_fn(tpad, emb.shape[0], emb.shape[1])(emb, tok_pad)[:n_gene]

    h = {
        "g": _gene_init(expr, emb_rows, mlp_W1, mlp_b1, mlp_W2, mlp_b2, gp_W, gp_b),
        "p": _scalar_init(jnp.zeros((N_PATH_PAD,), F32).at[:n_path].set(path_feat),
                          pp_W, pp_b),
        "v": _scalar_init(jnp.zeros((N_VIRT_PAD,), F32).at[:n_virt].set(virt_feat),
                          vp_W, vp_b),
    }
    nrows = {"g": n_gene, "p": N_PATH_PAD, "v": N_VIRT_PAD}
    rbs = {"g": 1000, "p": N_PATH_PAD, "v": N_VIRT_PAD}

    etypes = [
        ("g", "g", edge_g2g[0], edge_g2g[1]),
        ("g", "p", src_g2p, dst_g2p),
        ("p", "g", src_p2g, dst_p2g),
        ("p", "p", edge_p2p[0], edge_p2p[1]),
        ("v", "p", src_v2p, dst_v2p),
        ("p", "v", src_p2v, dst_p2v),
    ]
    padded = [_pad_edges(s, d, nrows[dk]) for (_, dk, s, d) in etypes]

    for l in range(NLAYERS):
        for t, (sk, dk, _, _) in enumerate(etypes):
            src_p, dstg_p, dsts_p = padded[t]
            if sk == dk:
                q, kk, vv = _proj(h[sk],
                                  jnp.stack([Wq[l, t], Wk[l, t], Wv[l, t]]),
                                  jnp.stack([bq[l, t], bk[l, t], bv[l, t]]),
                                  rbs[sk])
            else:
                (q,) = _proj(h[dk], Wq[l, t][None], bq[l, t][None], rbs[dk])
                kk, vv = _proj(h[sk],
                               jnp.stack([Wk[l, t], Wv[l, t]]),
                               jnp.stack([bk[l, t], bv[l, t]]),
                               rbs[sk])
            nd = nrows[dk]
            nd_sc = ((nd + 16 + NS * CB - 1) // (NS * CB)) * (NS * CB)
            p_out, p_den = _edge_pass(q, kk, vv, src_p, dstg_p, dsts_p, nd_sc,
                                      use_sc=2)
            h[dk] = _finish(h[dk], p_out, p_den, Wo[l, t],
                            bo[l, t], ln_g[l, t], ln_b[l, t], rbs[dk])

    return h["g"][:n_gene], h["p"][:n_path], h["v"][:n_virt]


# speedup vs baseline: 4.0683x; 1.3495x over previous
import functools

import numpy as np
import jax
import jax.numpy as jnp
from jax import lax
from jax.experimental import pallas as pl
from jax.experimental.pallas import tpu as pltpu
from jax.experimental.pallas import tpu_sc as plsc

F32 = jnp.float32
DIM = 128
H = 8
DK = 16
NLAYERS = 2
NC = 2
NS = 16
NW = NC * NS
CB = 32
NPACK = CB // 8

N_GENE = 10000
N_PATH_PAD = 512
N_VIRT_PAD = 16


def _proj_body(x_ref, w_ref, b_ref, *o_refs):
    x = x_ref[...]
    for p, o in enumerate(o_refs):
        o[...] = jnp.dot(x, w_ref[p], preferred_element_type=F32) + b_ref[p]


def _proj(x, ws, bs, rb):
    n = x.shape[0]
    P = ws.shape[0]
    grid = n // rb
    return pl.pallas_call(
        _proj_body,
        grid=(grid,),
        in_specs=[
            pl.BlockSpec((rb, DIM), lambda i: (i, 0)),
            pl.BlockSpec((P, DIM, DIM), lambda i: (0, 0, 0)),
            pl.BlockSpec((P, 1, DIM), lambda i: (0, 0, 0)),
        ],
        out_specs=[pl.BlockSpec((rb, DIM), lambda i: (i, 0))] * P,
        out_shape=[jax.ShapeDtypeStruct((n, DIM), F32)] * P,
    )(x, ws, bs.reshape(P, 1, DIM))


def _gene_init_body(expr_ref, embr_ref, w1_ref, b1_ref, w2_ref, b2_ref,
                    ga_ref, gb_ref, gbias_ref, o_ref):
    a = expr_ref[...] * w1_ref[...] + b1_ref[...]
    hx = 0.5 * a * (1.0 + lax.erf(a * np.float32(0.7071067811865476)))
    hx = jnp.dot(hx, w2_ref[...], preferred_element_type=F32) + b2_ref[...]
    o_ref[...] = (jnp.dot(hx, ga_ref[...], preferred_element_type=F32)
                  + jnp.dot(embr_ref[...], gb_ref[...], preferred_element_type=F32)
                  + gbias_ref[...])


def _gene_init(expr, emb_rows, w1, b1, w2, b2, gw, gb):
    n = expr.shape[0]
    rb = 1000
    grid = n // rb
    return pl.pallas_call(
        _gene_init_body,
        grid=(grid,),
        in_specs=[
            pl.BlockSpec((rb, 1), lambda i: (i, 0)),
            pl.BlockSpec((rb, 256), lambda i: (i, 0)),
            pl.BlockSpec((1, 256), lambda i: (0, 0)),
            pl.BlockSpec((1, 256), lambda i: (0, 0)),
            pl.BlockSpec((256, 256), lambda i: (0, 0)),
            pl.BlockSpec((1, 256), lambda i: (0, 0)),
            pl.BlockSpec((256, DIM), lambda i: (0, 0)),
            pl.BlockSpec((256, DIM), lambda i: (0, 0)),
            pl.BlockSpec((1, DIM), lambda i: (0, 0)),
        ],
        out_specs=pl.BlockSpec((rb, DIM), lambda i: (i, 0)),
        out_shape=jax.ShapeDtypeStruct((n, DIM), F32),
    )(expr, emb_rows, w1, b1.reshape(1, 256), w2, b2.reshape(1, 256),
      gw[:256], gw[256:], gb.reshape(1, DIM))


def _scalar_init_body(f_ref, w_ref, b_ref, o_ref):
    o_ref[...] = f_ref[...] * w_ref[...] + b_ref[...]


def _scalar_init(feat, w, b):
    n = feat.shape[0]
    return pl.pallas_call(
        _scalar_init_body,
        out_shape=jax.ShapeDtypeStruct((n, DIM), F32),
    )(feat.reshape(n, 1), w.reshape(1, DIM), b.reshape(1, DIM))


_R_HEAD = np.zeros((16, DIM), np.float32)
for _h in range(H):
    _R_HEAD[_h, _h * DK:(_h + 1) * DK] = 1.0


def _finish_body(h_ref, p0_ref, p1_ref, d0_ref, d1_ref, r_ref,
                 wo_ref, bo_ref, g_ref, b_ref, o_ref):
    den = jnp.dot(d0_ref[0] + d1_ref[0], r_ref[...],
                  preferred_element_type=F32)
    s = p0_ref[0] + p1_ref[0]
    attn = s / jnp.maximum(den, np.float32(1e-30))
    x = h_ref[...] + jnp.dot(attn, wo_ref[...], preferred_element_type=F32) + bo_ref[...]
    mu = jnp.mean(x, axis=-1, keepdims=True)
    xc = x - mu
    var = jnp.mean(xc * xc, axis=-1, keepdims=True)
    o_ref[...] = g_ref[...] * xc * lax.rsqrt(var + np.float32(1e-5)) + b_ref[...]


def _finish(h, p_out, p_den, wo, bo, g, b, rb):
    n = h.shape[0]
    grid = n // rb
    return pl.pallas_call(
        _finish_body,
        grid=(grid,),
        in_specs=[
            pl.BlockSpec((rb, DIM), lambda i: (i, 0)),
            pl.BlockSpec((1, rb, DIM), lambda i: (0, i, 0)),
            pl.BlockSpec((1, rb, DIM), lambda i: (1, i, 0)),
            pl.BlockSpec((1, rb, 16), lambda i: (0, i, 0)),
            pl.BlockSpec((1, rb, 16), lambda i: (1, i, 0)),
            pl.BlockSpec((16, DIM), lambda i: (0, 0)),
            pl.BlockSpec((DIM, DIM), lambda i: (0, 0)),
            pl.BlockSpec((1, DIM), lambda i: (0, 0)),
            pl.BlockSpec((1, DIM), lambda i: (0, 0)),
            pl.BlockSpec((1, DIM), lambda i: (0, 0)),
        ],
        out_specs=pl.BlockSpec((rb, DIM), lambda i: (i, 0)),
        out_shape=jax.ShapeDtypeStruct((n, DIM), F32),
    )(h, p_out, p_out, p_den, p_den, jnp.asarray(_R_HEAD),
      wo, bo.reshape(1, DIM), g.reshape(1, DIM), b.reshape(1, DIM))


def _shuffle16(x, p):
    dn = lax.GatherDimensionNumbers(offset_dims=(), collapsed_slice_dims=(0,),
                                    start_index_map=(0,))
    return lax.gather(x, p[:, None], dn, slice_sizes=(1,),
                      mode=lax.GatherScatterMode.PROMISE_IN_BOUNDS)


def _sc_mesh():
    return plsc.VectorSubcoreMesh(core_axis_name="c", subcore_axis_name="s",
                                  num_cores=NC, num_subcores=NS)


@functools.lru_cache(maxsize=None)
def _edge_pass_fn(epad, nd_sc, steps):
    rows_t = nd_sc // NS
    rows_td = rows_t // 8
    nsteps_zd = rows_t // CB

    idx_t = pltpu.VMEM((CB,), jnp.int32)
    row_t = pltpu.VMEM((CB, DIM), F32)

    @functools.partial(
        pl.kernel,
        out_type=[jax.ShapeDtypeStruct((NC, nd_sc, DIM), F32),
                  jax.ShapeDtypeStruct((NC, nd_sc, 16), F32)],
        mesh=_sc_mesh(),
        scratch_types=(
            [idx_t, idx_t, idx_t, idx_t, row_t, row_t, row_t] * 2 +
            [pltpu.VMEM((CB, 16), F32),
             pltpu.VMEM_SHARED((nd_sc, DIM), F32),
             pltpu.VMEM_SHARED((nd_sc // 8, DIM), F32),
             pltpu.SemaphoreType.DMA,
             pltpu.SemaphoreType.DMA]
        ),
    )
    def kern(q_hbm, k_hbm, v_hbm, src_hbm, dst_hbm, dsts_hbm,
             out_hbm, den_hbm,
             srcA, dstA, dstsA, d2A, kA, qA, vA,
             srcB, dstB, dstsB, d2B, kB, qB, vB,
             dvo, acc_o, acc_d, semI, semG):
        cid = lax.axis_index("c")
        sid = lax.axis_index("s")
        wid = sid * NC + cid
        r0 = sid * rows_t
        r0d = sid * rows_td
        bufA = (srcA, dstA, dstsA, d2A, kA, qA, vA)
        bufB = (srcB, dstB, dstsB, d2B, kB, qB, vB)

        zvec = jnp.zeros((16,), F32)

        def zfill(j, c):
            kA[j >> 3, pl.ds((j & 7) * 16, 16)] = zvec
            return c

        lax.fori_loop(0, CB * 8, zfill, 0)

        def zstep(j, c):
            pltpu.sync_copy(kA, acc_o.at[pl.ds(r0 + j * CB, CB)])
            pltpu.sync_copy(kA.at[pl.ds(0, NPACK)],
                            acc_d.at[pl.ds(r0d + j * NPACK, NPACK)])
            return c

        lax.fori_loop(0, nsteps_zd, zstep, 0)
        plsc.subcore_barrier()

        wbase = wid * (steps * CB)

        def issue_idx(t, buf):
            src_v, dst_v, dsts_v = buf[0], buf[1], buf[2]
            base = wbase + t * CB
            c1 = pltpu.async_copy(src_hbm.at[pl.ds(base, CB)], src_v, semI)
            c2 = pltpu.async_copy(dst_hbm.at[pl.ds(base, CB)], dst_v, semI)
            c3 = pltpu.async_copy(dsts_hbm.at[pl.ds(base, CB)], dsts_v, semI)
            return c1, c2, c3

        def d2_compute(buf):
            dsts_v, d2_v = buf[2], buf[3]
            for g in range(CB // 16):
                d2_v[pl.ds(g * 16, 16)] = lax.shift_right_logical(
                    dsts_v[pl.ds(g * 16, 16)], 3)

        def issue_gather(buf):
            src_v, dst_v, k_v, q_v, v_v = buf[0], buf[1], buf[4], buf[5], buf[6]
            pltpu.async_copy(k_hbm.at[src_v], k_v, semG)
            pltpu.async_copy(q_hbm.at[dst_v], q_v, semG)
            pltpu.async_copy(v_hbm.at[src_v], v_v, semG)

        def drain_gather(buf):
            for r_v in (buf[4], buf[5], buf[6]):
                pltpu.make_async_copy(out_hbm.at[0, pl.ds(0, CB)], r_v,
                                      semG).wait()

        lane = lax.iota(jnp.int32, 16)
        perms = [jnp.bitwise_xor(lane, sh) for sh in (8, 4, 2, 1)]
        seven = jnp.full((16,), 7, jnp.int32)

        def compute_scatter(buf):
            dsts_v, d2_v, k_v, q_v, v_v = buf[2], buf[3], buf[4], buf[5], buf[6]

            @plsc.parallel_loop(0, CB, 1)
            def edge(e):
                den_row = jnp.zeros((16,), F32)
                for h in range(H):
                    kvec = k_v[e, pl.ds(h * DK, DK)]
                    qvec = q_v[e, pl.ds(h * DK, DK)]
                    x = kvec * qvec
                    for p in perms:
                        x = x + _shuffle16(x, p)
                    ev = jnp.exp(x * np.float32(0.25))
                    v_v[e, pl.ds(h * DK, DK)] = v_v[e, pl.ds(h * DK, DK)] * ev
                    den_row = den_row + jnp.where(lane == h, ev, np.float32(0.0))
                dgrp = dsts_v[pl.ds((e >> 4) * 16, 16)]
                dlow = _shuffle16(dgrp, jnp.full((16,), e & 15, jnp.int32)) & seven
                df = dlow.astype(F32)
                for c in range(8):
                    w = jnp.maximum(np.float32(0.0),
                                    np.float32(1.0) - jnp.abs(df - np.float32(c)))
                    q_v[e, pl.ds(c * 16, 16)] = den_row * w

            pltpu.sync_copy(v_v, acc_o.at[dsts_v], add=True)
            pltpu.sync_copy(q_v, acc_d.at[d2_v], add=True)

        i1, i2, i3 = issue_idx(0, bufA)
        i1.wait(); i2.wait(); i3.wait()
        d2_compute(bufA)
        issue_gather(bufA)

        def pairstep(tp, carry):
            t0 = tp * 2
            for b in range(2):
                cur, nxt = (bufA, bufB) if b == 0 else (bufB, bufA)
                t = t0 + b
                tn = jnp.minimum(t + 1, steps - 1)
                i1, i2, i3 = issue_idx(tn, nxt)
                drain_gather(cur)
                i1.wait(); i2.wait(); i3.wait()
                d2_compute(nxt)
                issue_gather(nxt)
                compute_scatter(cur)
            return carry

        lax.fori_loop(0, steps // 2, pairstep, 0)
        drain_gather(bufA)
        plsc.subcore_barrier()

        def dstep(j, c):
            pltpu.sync_copy(acc_o.at[pl.ds(r0 + j * CB, CB)], kA)
            pltpu.sync_copy(kA, out_hbm.at[cid, pl.ds(r0 + j * CB, CB)])
            pltpu.sync_copy(acc_d.at[pl.ds(r0d + j * NPACK, NPACK)],
                            qA.at[pl.ds(0, NPACK)])
            for j8 in range(NPACK):
                for j2 in range(8):
                    dvo[j8 * 8 + j2, pl.ds(0, 16)] = qA[j8, pl.ds(j2 * 16, 16)]
            pltpu.sync_copy(dvo, den_hbm.at[cid, pl.ds(r0 + j * CB, CB)])
            return c

        lax.fori_loop(0, nsteps_zd, dstep, 0)

    return kern


@functools.lru_cache(maxsize=None)
def _edge_msg_fn(epad, steps):
    @functools.partial(
        pl.kernel,
        out_type=[jax.ShapeDtypeStruct((epad, DIM), F32),
                  jax.ShapeDtypeStruct((epad, 16), F32)],
        mesh=_sc_mesh(),
        scratch_types=[
            pltpu.VMEM((CB,), jnp.int32),
            pltpu.VMEM((CB,), jnp.int32),
            pltpu.VMEM((CB, DIM), F32),
            pltpu.VMEM((CB, DIM), F32),
            pltpu.VMEM((CB, DIM), F32),
            pltpu.VMEM((CB, 16), F32),
            pltpu.SemaphoreType.DMA,
        ],
    )
    def kern(q_hbm, k_hbm, v_hbm, src_hbm, dst_hbm, msg_hbm, den_hbm,
             srcv, dstv, kv, qv, vv, dv, sem):
        cid = lax.axis_index("c")
        sid = lax.axis_index("s")
        wid = sid * NC + cid
        wbase = wid * (steps * CB)

        def step(t, carry):
            base = wbase + t * CB
            pltpu.sync_copy(src_hbm.at[pl.ds(base, CB)], srcv)
            pltpu.sync_copy(dst_hbm.at[pl.ds(base, CB)], dstv)
            pltpu.async_copy(k_hbm.at[srcv], kv, sem).wait()
            pltpu.async_copy(q_hbm.at[dstv], qv, sem).wait()
            pltpu.async_copy(v_hbm.at[srcv], vv, sem).wait()

            lane = lax.iota(jnp.int32, 16)
            perms = [jnp.bitwise_xor(lane, sh) for sh in (8, 4, 2, 1)]

            def edge(e, c2):
                den_row = jnp.zeros((16,), F32)
                for h in range(H):
                    kvec = kv[e, pl.ds(h * DK, DK)]
                    qvec = qv[e, pl.ds(h * DK, DK)]
                    x = kvec * qvec
                    for p in perms:
                        x = x + _shuffle16(x, p)
                    ev = jnp.exp(x * np.float32(0.25))
                    vv[e, pl.ds(h * DK, DK)] = vv[e, pl.ds(h * DK, DK)] * ev
                    den_row = den_row + jnp.where(lane == h, ev, np.float32(0.0))
                dv[e, pl.ds(0, 16)] = den_row
                return c2

            lax.fori_loop(0, CB, edge, 0)
            pltpu.sync_copy(vv, msg_hbm.at[pl.ds(base, CB)])
            pltpu.sync_copy(dv, den_hbm.at[pl.ds(base, CB)])
            return carry

        lax.fori_loop(0, steps, step, 0)

    return kern


def _edge_pass(q, k, v, src_pad, dstg_pad, dsts_pad, nd_sc, use_sc):
    if use_sc == 1:
        epad = src_pad.shape[0]
        steps = epad // (NW * CB)
        msg, den = _edge_msg_fn(epad, steps)(q, k, v, src_pad, dstg_pad)
        out = jax.ops.segment_sum(msg, dsts_pad, num_segments=nd_sc)
        dsum = jax.ops.segment_sum(den, dsts_pad, num_segments=nd_sc)
        return (jnp.stack([out, jnp.zeros_like(out)]),
                jnp.stack([dsum, jnp.zeros_like(dsum)]))
    if not use_sc:
        score = jnp.einsum("ehd,ehd->eh", k[src_pad].reshape(-1, H, DK),
                           q[dstg_pad].reshape(-1, H, DK)) * 0.25
        ev = jnp.exp(score)
        den = jax.ops.segment_sum(ev, dsts_pad, num_segments=nd_sc)
        msg = v[src_pad].reshape(-1, H, DK) * ev[:, :, None]
        out = jax.ops.segment_sum(msg, dsts_pad, num_segments=nd_sc).reshape(-1, DIM)
        zo = jnp.zeros_like(out)
        zd = jnp.zeros((nd_sc, 16), F32)
        return (jnp.stack([out, zo]),
                jnp.stack([jnp.concatenate([den, jnp.zeros((nd_sc, 8), F32)], 1), zd]))
    epad = src_pad.shape[0]
    steps = epad // (NW * CB)
    fn = _edge_pass_fn(epad, nd_sc, steps)
    return fn(q, k, v, src_pad, dstg_pad, dsts_pad)


@functools.lru_cache(maxsize=None)
def _emb_gather_fn(tpad, nvocab, dmodel):
    steps = tpad // (NW * CB)

    @functools.partial(
        pl.kernel,
        out_type=jax.ShapeDtypeStruct((tpad, dmodel), F32),
        mesh=_sc_mesh(),
        scratch_types=[
            pltpu.VMEM((CB,), jnp.int32),
            pltpu.VMEM((CB, dmodel), F32),
            pltpu.SemaphoreType.DMA,
        ],
    )
    def kern(emb_hbm, tok_hbm, out_hbm, idxv, rows, sem):
        cid = lax.axis_index("c")
        sid = lax.axis_index("s")
        wid = sid * NC + cid
        wbase = wid * (steps * CB)

        def step(t, carry):
            base = wbase + t * CB
            pltpu.sync_copy(tok_hbm.at[pl.ds(base, CB)], idxv)
            pltpu.async_copy(emb_hbm.at[idxv], rows, sem).wait()
            pltpu.sync_copy(rows, out_hbm.at[pl.ds(base, CB)])
            return carry

        lax.fori_loop(0, steps, step, 0)

    return kern


def _pad_edges(src, dst, trash_row):
    e_real = src.shape[0]
    chunk = NW * CB
    epad = ((e_real + chunk - 1) // chunk) * chunk
    src_p = jnp.zeros((epad,), jnp.int32).at[:e_real].set(src.astype(jnp.int32))
    dstg_p = jnp.zeros((epad,), jnp.int32).at[:e_real].set(dst.astype(jnp.int32))
    dsts_p = jnp.full((epad,), trash_row, jnp.int32).at[:e_real].set(dst.astype(jnp.int32))
    return src_p, dstg_p, dsts_p


def kernel(expr, token, path_feat, virt_feat, edge_g2g, src_g2p, dst_g2p,
           src_p2g, dst_p2g, edge_p2p, src_v2p, dst_v2p, src_p2v, dst_p2v,
           emb, mlp_W1, mlp_b1, mlp_W2, mlp_b2, gp_W, gp_b, pp_W, pp_b,
           vp_W, vp_b, Wq, bq, Wk, bk, Wv, bv, Wo, bo, ln_g, ln_b):
    n_gene = expr.shape[0]
    n_path = path_feat.shape[0]
    n_virt = virt_feat.shape[0]

    tpad = ((n_gene + NW * CB - 1) // (NW * CB)) * (NW * CB)
    tok_pad = jnp.zeros((tpad,), jnp.int32).at[:n_gene].set(token.astype(jnp.int32))
    emb_rows = _emb_gather_fn(tpad, emb.shape[0], emb.shape[1])(emb, tok_pad)[:n_gene]

    h = {
        "g": _gene_init(expr, emb_rows, mlp_W1, mlp_b1, mlp_W2, mlp_b2, gp_W, gp_b),
        "p": _scalar_init(jnp.zeros((N_PATH_PAD,), F32).at[:n_path].set(path_feat),
                          pp_W, pp_b),
        "v": _scalar_init(jnp.zeros((N_VIRT_PAD,), F32).at[:n_virt].set(virt_feat),
                          vp_W, vp_b),
    }
    nrows = {"g": n_gene, "p": N_PATH_PAD, "v": N_VIRT_PAD}
    rbs = {"g": 1000, "p": N_PATH_PAD, "v": N_VIRT_PAD}

    etypes = [
        ("g", "g", edge_g2g[0], edge_g2g[1]),
        ("g", "p", src_g2p, dst_g2p),
        ("p", "g", src_p2g, dst_p2g),
        ("p", "p", edge_p2p[0], edge_p2p[1]),
        ("v", "p", src_v2p, dst_v2p),
        ("p", "v", src_p2v, dst_p2v),
    ]
    padded = [_pad_edges(s, d, nrows[dk]) for (_, dk, s, d) in etypes]

    for l in range(NLAYERS):
        for t, (sk, dk, _, _) in enumerate(etypes):
            src_p, dstg_p, dsts_p = padded[t]
            if sk == dk:
                q, kk, vv = _proj(h[sk],
                                  jnp.stack([Wq[l, t], Wk[l, t], Wv[l, t]]),
                                  jnp.stack([bq[l, t], bk[l, t], bv[l, t]]),
                                  rbs[sk])
            else:
                (q,) = _proj(h[dk], Wq[l, t][None], bq[l, t][None], rbs[dk])
                kk, vv = _proj(h[sk],
                               jnp.stack([Wk[l, t], Wv[l, t]]),
                               jnp.stack([bk[l, t], bv[l, t]]),
                               rbs[sk])
            nd = nrows[dk]
            nd_sc = ((nd + 16 + NS * CB - 1) // (NS * CB)) * (NS * CB)
            p_out, p_den = _edge_pass(q, kk, vv, src_p, dstg_p, dsts_p, nd_sc,
                                      use_sc=2)
            h[dk] = _finish(h[dk], p_out, p_den, Wo[l, t],
                            bo[l, t], ln_g[l, t], ln_b[l, t], rbs[dk])

    return h["g"][:n_gene], h["p"][:n_path], h["v"][:n_virt]
